# baseline (speedup 1.0000x reference)
import functools

import numpy as np
import jax
import jax.numpy as jnp
from jax.experimental import pallas as pl
from jax.experimental.pallas import tpu as pltpu


def _conv_pool_kernel(w_ref, x_ref, comp_ref, scale_ref, shift_ref, pool_ref,
                      o_ref, wt_ref, a3c_ref,
                      *, n_im, kh_taps, kw_taps, cin, wp, ohn):
    kh = pl.program_id(1)
    nwc = ohn * 12
    kc = kw_taps * cin

    @pl.when(kh == 0)
    def _init():
        for n in range(n_im):
            for kw in range(kw_taps):
                a3c_ref[n, kw * cin:(kw + 1) * cin, :] = jnp.dot(
                    x_ref[n], comp_ref[kw],
                    preferred_element_type=jnp.float32).astype(jnp.bfloat16)

    wt_ref[:, pl.ds(kh * kc, kc)] = jnp.concatenate(
        [w_ref[kw].astype(jnp.bfloat16) for kw in range(kw_taps)], axis=1)

    @pl.when(kh == kh_taps - 1)
    def _compute():
        ys = []
        for n in range(n_im):
            acc = jnp.dot(wt_ref[:, 0:kc], a3c_ref[n, :, 0:nwc],
                          preferred_element_type=jnp.float32)
            for kh_s in range(1, kh_taps):
                acc += jnp.dot(wt_ref[:, kh_s * kc:(kh_s + 1) * kc],
                               a3c_ref[n, :, kh_s * 12:kh_s * 12 + nwc],
                               preferred_element_type=jnp.float32)
            acc = acc * scale_ref[:, 0:1]
            m = jnp.maximum(
                jnp.maximum(acc[:, 0:nwc - 13], acc[:, 1:nwc - 12]),
                jnp.maximum(acc[:, 12:nwc - 1], acc[:, 13:nwc]))
            m = m + shift_ref[:, 0:1]
            y = jnp.where(m >= 0.0, m, 0.01 * m)
            ys.append(jnp.pad(y.astype(jnp.bfloat16), ((0, 0), (0, 13))))
        y = jnp.concatenate(ys, axis=1)
        o_ref[...] = jnp.dot(y, pool_ref[...],
                             preferred_element_type=jnp.float32)


@functools.partial(jax.jit, static_argnames=("kernel_size", "padding"))
def _forward(x, w, b, gamma, beta, running_mean, running_var,
             *, kernel_size=7, padding=2, eps=1e-5):
    N, Cin, H, W = x.shape
    Cout = w.shape[0]
    KH = KW = kernel_size
    Hp, Wp = H + 2 * padding, W + 2 * padding
    OHn = 12
    AH = AW = 3
    P = N * AH * AW
    CB = Cout // 2

    scale = (gamma * jax.lax.rsqrt(running_var + eps)).astype(jnp.float32)
    shift = (beta + scale * (b - running_mean)).astype(jnp.float32)
    scale_col = jnp.broadcast_to(scale[:, None], (Cout, 128))
    shift_col = jnp.broadcast_to(shift[:, None], (Cout, 128))

    wq = jnp.transpose(w, (2, 3, 0, 1)).reshape(KH * KW, Cout, Cin)

    xp = jnp.pad(x, ((0, 0), (0, 0), (padding, padding), (padding, padding)))
    xp = xp.reshape(N, Cin, Hp * Wp).astype(jnp.bfloat16)
    xp = jnp.pad(xp, ((0, 0), (0, 0), (0, 512 - Hp * Wp)))

    comp = np.zeros((KW, 512, 256), np.float32)
    for kw in range(KW):
        for h in range(OHn + KH - 1):
            for ow in range(12):
                comp[kw, h * Wp + ow + kw, h * 12 + ow] = 1.0
    comp = jnp.asarray(comp, dtype=jnp.bfloat16)

    pool = np.zeros((N * 144, 128), np.float32)
    for n in range(N):
        for ah in range(AH):
            for aw in range(AW):
                dst = n * AH * AW + ah * AW + aw
                for da in range(2):
                    for db in range(2):
                        src = n * 144 + 2 * (2 * ah + da) * 12 + 2 * (2 * aw + db)
                        pool[src, dst] = 0.25
    pool = jnp.asarray(pool, dtype=jnp.bfloat16)

    flops = 2 * Cout * KH * KW * Cin * N * 144 + 2 * Cout * N * 144 * 128
    bytes_accessed = 4 * Cout * KH * KW * Cin + 2 * (N * Cin * 512 + N * 144 * 128) + 4 * Cout * 256
    out2d = pl.pallas_call(
        functools.partial(_conv_pool_kernel, n_im=N, kh_taps=KH, kw_taps=KW,
                          cin=Cin, wp=Wp, ohn=OHn),
        out_shape=jax.ShapeDtypeStruct((Cout, 128), jnp.float32),
        grid=(2, KH),
        in_specs=[
            pl.BlockSpec((KW, CB, Cin), lambda i, k: (k, i, 0)),
            pl.BlockSpec((N, Cin, 512), lambda i, k: (0, 0, 0)),
            pl.BlockSpec((KW, 512, 256), lambda i, k: (0, 0, 0)),
            pl.BlockSpec((CB, 128), lambda i, k: (i, 0)),
            pl.BlockSpec((CB, 128), lambda i, k: (i, 0)),
            pl.BlockSpec((N * 144, 128), lambda i, k: (0, 0)),
        ],
        out_specs=pl.BlockSpec((CB, 128), lambda i, k: (i, 0)),
        scratch_shapes=[pltpu.VMEM((CB, KH * KW * Cin), jnp.bfloat16),
                        pltpu.VMEM((N, KW * Cin, 256), jnp.bfloat16)],
        compiler_params=pltpu.CompilerParams(
            dimension_semantics=("parallel", "arbitrary")),
        cost_estimate=pl.CostEstimate(flops=flops, transcendentals=0,
                                      bytes_accessed=bytes_accessed),
    )(wq, xp, comp, scale_col, shift_col, pool)

    return jnp.transpose(out2d[:, :P].reshape(Cout, N, AH, AW), (1, 0, 2, 3))


def kernel(x, w, b, gamma, beta, running_mean, running_var):
    return _forward(x, w, b, gamma, beta, running_mean, running_var,
                    kernel_size=7, padding=2)


# speedup vs baseline: 1.0894x; 1.0894x over previous
import functools

import numpy as np
import jax
import jax.numpy as jnp
from jax.experimental import pallas as pl
from jax.experimental.pallas import tpu as pltpu


def _conv_pool_kernel(w_ref, x_ref, comp_ref, scale_ref, shift_ref, pool_ref,
                      o_ref, acc_ref, a3c_ref,
                      *, n_im, kh_taps, kw_taps, cin, wp, ohn):
    kh = pl.program_id(1)
    nwc = ohn * 12

    @pl.when(kh == 0)
    def _init():
        acc_ref[...] = jnp.zeros_like(acc_ref)
        for n in range(n_im):
            for kw in range(kw_taps):
                a3c_ref[n, kw * cin:(kw + 1) * cin, :] = jnp.dot(
                    x_ref[n], comp_ref[kw],
                    preferred_element_type=jnp.float32).astype(jnp.bfloat16)

    wt = jnp.concatenate([w_ref[kw].astype(jnp.bfloat16)
                          for kw in range(kw_taps)], axis=1)
    for kh_s in range(kh_taps):
        @pl.when(kh == kh_s)
        def _tap(s=kh_s * 12):
            for n in range(n_im):
                acc_ref[n, :, 0:nwc] += jnp.dot(
                    wt, a3c_ref[n, :, s:s + nwc],
                    preferred_element_type=jnp.float32)

    @pl.when(kh == kh_taps - 1)
    def _epilogue():
        ys = []
        for n in range(n_im):
            acc = acc_ref[n, :, 0:nwc] * scale_ref[:, 0:1]
            m = jnp.maximum(
                jnp.maximum(acc[:, 0:nwc - 13], acc[:, 1:nwc - 12]),
                jnp.maximum(acc[:, 12:nwc - 1], acc[:, 13:nwc]))
            m = m + shift_ref[:, 0:1]
            y = jnp.where(m >= 0.0, m, 0.01 * m)
            ys.append(jnp.pad(y.astype(jnp.bfloat16), ((0, 0), (0, 13))))
        y = jnp.concatenate(ys, axis=1)
        o_ref[...] = jnp.dot(y, pool_ref[...],
                             preferred_element_type=jnp.float32)


@functools.partial(jax.jit, static_argnames=("kernel_size", "padding"))
def _forward(x, w, b, gamma, beta, running_mean, running_var,
             *, kernel_size=7, padding=2, eps=1e-5):
    N, Cin, H, W = x.shape
    Cout = w.shape[0]
    KH = KW = kernel_size
    Hp, Wp = H + 2 * padding, W + 2 * padding
    OHn = 12
    AH = AW = 3
    P = N * AH * AW
    CB = Cout // 2

    scale = (gamma * jax.lax.rsqrt(running_var + eps)).astype(jnp.float32)
    shift = (beta + scale * (b - running_mean)).astype(jnp.float32)
    scale_col = jnp.broadcast_to(scale[:, None], (Cout, 128))
    shift_col = jnp.broadcast_to(shift[:, None], (Cout, 128))

    wq = jnp.transpose(w, (2, 3, 0, 1)).reshape(KH * KW, Cout, Cin)

    xp = jnp.pad(x, ((0, 0), (0, 0), (padding, padding), (padding, padding)))
    xp = xp.reshape(N, Cin, Hp * Wp).astype(jnp.bfloat16)
    xp = jnp.pad(xp, ((0, 0), (0, 0), (0, 512 - Hp * Wp)))

    comp = np.zeros((KW, 512, 256), np.float32)
    for kw in range(KW):
        for h in range(OHn + KH - 1):
            for ow in range(12):
                comp[kw, h * Wp + ow + kw, h * 12 + ow] = 1.0
    comp = jnp.asarray(comp, dtype=jnp.bfloat16)

    pool = np.zeros((N * 144, 128), np.float32)
    for n in range(N):
        for ah in range(AH):
            for aw in range(AW):
                dst = n * AH * AW + ah * AW + aw
                for da in range(2):
                    for db in range(2):
                        src = n * 144 + 2 * (2 * ah + da) * 12 + 2 * (2 * aw + db)
                        pool[src, dst] = 0.25
    pool = jnp.asarray(pool, dtype=jnp.bfloat16)

    flops = 2 * Cout * KH * KW * Cin * N * 144 + 2 * Cout * N * 144 * 128
    bytes_accessed = 4 * Cout * KH * KW * Cin + 2 * (N * Cin * 512 + N * 144 * 128) + 4 * Cout * 256
    out2d = pl.pallas_call(
        functools.partial(_conv_pool_kernel, n_im=N, kh_taps=KH, kw_taps=KW,
                          cin=Cin, wp=Wp, ohn=OHn),
        out_shape=jax.ShapeDtypeStruct((Cout, 128), jnp.float32),
        grid=(2, KH),
        in_specs=[
            pl.BlockSpec((KW, CB, Cin), lambda i, k: (k, i, 0)),
            pl.BlockSpec((N, Cin, 512), lambda i, k: (0, 0, 0)),
            pl.BlockSpec((KW, 512, 256), lambda i, k: (0, 0, 0)),
            pl.BlockSpec((CB, 128), lambda i, k: (i, 0)),
            pl.BlockSpec((CB, 128), lambda i, k: (i, 0)),
            pl.BlockSpec((N * 144, 128), lambda i, k: (0, 0)),
        ],
        out_specs=pl.BlockSpec((CB, 128), lambda i, k: (i, 0)),
        scratch_shapes=[pltpu.VMEM((N, CB, 256), jnp.float32),
                        pltpu.VMEM((N, KW * Cin, 256), jnp.bfloat16)],
        compiler_params=pltpu.CompilerParams(
            dimension_semantics=("parallel", "arbitrary")),
        cost_estimate=pl.CostEstimate(flops=flops, transcendentals=0,
                                      bytes_accessed=bytes_accessed),
    )(wq, xp, comp, scale_col, shift_col, pool)

    return jnp.transpose(out2d[:, :P].reshape(Cout, N, AH, AW), (1, 0, 2, 3))


def kernel(x, w, b, gamma, beta, running_mean, running_var):
    return _forward(x, w, b, gamma, beta, running_mean, running_var,
                    kernel_size=7, padding=2)


# speedup vs baseline: 1.1998x; 1.1013x over previous
import functools

import numpy as np
import jax
import jax.numpy as jnp
from jax.experimental import pallas as pl
from jax.experimental.pallas import tpu as pltpu


def _conv_pool_kernel(w_ref, x_ref, comp_ref, scale_ref, shift_ref, pool_ref,
                      o_ref, acc_ref, a3c_ref,
                      *, n_im, kh_taps, kw_taps, cin, wp, ohn):
    kh = pl.program_id(1)
    nwc = ohn * 12

    @pl.when(kh == 0)
    def _init():
        acc_ref[...] = jnp.zeros_like(acc_ref)
        for n in range(n_im):
            cx = x_ref[n].astype(jnp.bfloat16)
            for kw in range(kw_taps):
                a3c_ref[n, kw * cin:(kw + 1) * cin, :] = jnp.dot(
                    cx, comp_ref[kw],
                    preferred_element_type=jnp.float32).astype(jnp.bfloat16)

    wt = jnp.concatenate([w_ref[kw].astype(jnp.bfloat16)
                          for kw in range(kw_taps)], axis=1)
    for kh_s in range(kh_taps):
        @pl.when(kh == kh_s)
        def _tap(s=kh_s * 12):
            for n in range(n_im):
                acc_ref[n, :, 0:nwc] += jnp.dot(
                    wt, a3c_ref[n, :, s:s + nwc],
                    preferred_element_type=jnp.float32)

    @pl.when(kh == kh_taps - 1)
    def _epilogue():
        ys = []
        for n in range(n_im):
            acc = acc_ref[n, :, 0:nwc] * scale_ref[:, 0:1]
            m = jnp.maximum(
                jnp.maximum(acc[:, 0:nwc - 13], acc[:, 1:nwc - 12]),
                jnp.maximum(acc[:, 12:nwc - 1], acc[:, 13:nwc]))
            m = m + shift_ref[:, 0:1]
            y = jnp.where(m >= 0.0, m, 0.01 * m)
            ys.append(jnp.pad(y.astype(jnp.bfloat16), ((0, 0), (0, 13))))
        y = jnp.concatenate(ys, axis=1)
        o_ref[...] = jnp.dot(y, pool_ref[...],
                             preferred_element_type=jnp.float32)


@functools.partial(jax.jit, static_argnames=("kernel_size", "padding"))
def _forward(x, w, b, gamma, beta, running_mean, running_var,
             *, kernel_size=7, padding=2, eps=1e-5):
    N, Cin, H, W = x.shape
    Cout = w.shape[0]
    KH = KW = kernel_size
    Hp, Wp = H + 2 * padding, W + 2 * padding
    OHn = 12
    AH = AW = 3
    P = N * AH * AW
    CB = Cout // 2

    scale = (gamma * jax.lax.rsqrt(running_var + eps)).astype(jnp.float32)
    shift = (beta + scale * (b - running_mean)).astype(jnp.float32)
    scale_col = jnp.broadcast_to(scale[:, None], (Cout, 128))
    shift_col = jnp.broadcast_to(shift[:, None], (Cout, 128))

    wq = jnp.transpose(w, (2, 3, 0, 1)).reshape(KH * KW, Cout, Cin)

    xr = x.reshape(N, Cin, H * W)

    comp = np.zeros((KW, H * W, 256), np.float32)
    for kw in range(KW):
        for h2 in range(OHn + KH - 1):
            for ow in range(12):
                h = h2 - padding
                iw = ow + kw - padding
                if 0 <= h < H and 0 <= iw < W:
                    comp[kw, h * W + iw, h2 * 12 + ow] = 1.0
    comp = jnp.asarray(comp, dtype=jnp.bfloat16)

    pool = np.zeros((N * 144, 128), np.float32)
    for n in range(N):
        for ah in range(AH):
            for aw in range(AW):
                dst = n * AH * AW + ah * AW + aw
                for da in range(2):
                    for db in range(2):
                        src = n * 144 + 2 * (2 * ah + da) * 12 + 2 * (2 * aw + db)
                        pool[src, dst] = 0.25
    pool = jnp.asarray(pool, dtype=jnp.bfloat16)

    flops = 2 * Cout * KH * KW * Cin * N * 144 + 2 * Cout * N * 144 * 128
    bytes_accessed = 4 * Cout * KH * KW * Cin + 2 * (N * Cin * 512 + N * 144 * 128) + 4 * Cout * 256
    out2d = pl.pallas_call(
        functools.partial(_conv_pool_kernel, n_im=N, kh_taps=KH, kw_taps=KW,
                          cin=Cin, wp=Wp, ohn=OHn),
        out_shape=jax.ShapeDtypeStruct((Cout, 128), jnp.float32),
        grid=(2, KH),
        in_specs=[
            pl.BlockSpec((KW, CB, Cin), lambda i, k: (k, i, 0)),
            pl.BlockSpec((N, Cin, H * W), lambda i, k: (0, 0, 0)),
            pl.BlockSpec((KW, H * W, 256), lambda i, k: (0, 0, 0)),
            pl.BlockSpec((CB, 128), lambda i, k: (i, 0)),
            pl.BlockSpec((CB, 128), lambda i, k: (i, 0)),
            pl.BlockSpec((N * 144, 128), lambda i, k: (0, 0)),
        ],
        out_specs=pl.BlockSpec((CB, 128), lambda i, k: (i, 0)),
        scratch_shapes=[pltpu.VMEM((N, CB, 256), jnp.float32),
                        pltpu.VMEM((N, KW * Cin, 256), jnp.bfloat16)],
        compiler_params=pltpu.CompilerParams(
            dimension_semantics=("parallel", "arbitrary")),
        cost_estimate=pl.CostEstimate(flops=flops, transcendentals=0,
                                      bytes_accessed=bytes_accessed),
    )(wq, xr, comp, scale_col, shift_col, pool)

    return jnp.transpose(out2d[:, :P].reshape(Cout, N, AH, AW), (1, 0, 2, 3))


def kernel(x, w, b, gamma, beta, running_mean, running_var):
    return _forward(x, w, b, gamma, beta, running_mean, running_var,
                    kernel_size=7, padding=2)
